# baseline (speedup 1.0000x reference)
import functools

import jax
import jax.numpy as jnp
from jax import lax
from jax.experimental import pallas as pl
from jax.experimental.pallas import tpu as pltpu
from jax.experimental.pallas import tpu_sc as plsc

_INTERPRET = False

Q = 64
D = 32
K = 8
CHUNK = 16384
SEG = 512
SEGS = CHUNK // SEG

_NEG_INF = float("-inf")
_I32_MAX = 2**31 - 1


def _stage_a_body(q_ref, k_ref, s_ref, b_ref, *, m_total):
    i = pl.program_id(0)
    q = q_ref[...]
    k = k_ref[...]
    s = jax.lax.dot_general(
        q, k, dimension_numbers=(((1,), (1,)), ((), ())),
        preferred_element_type=jnp.float32)
    col = i * CHUNK + jax.lax.broadcasted_iota(jnp.int32, (Q, CHUNK), 1)
    s = jnp.where(col < m_total, s, _NEG_INF)
    s3 = s.reshape(Q, SEGS, SEG)
    s_ref[...] = jnp.transpose(s3, (1, 0, 2)).reshape(SEGS * Q, SEG)
    b_ref[0] = jnp.max(s3, axis=2)


def _stage_b_body(bm_ref, bid_ref, *, nb):
    bm = bm_ref[...]
    cols = jax.lax.broadcasted_iota(jnp.int32, (Q, nb), 1)
    ids = []
    for _ in range(K):
        m = jnp.max(bm, axis=1, keepdims=True)
        pick = jnp.min(jnp.where(bm == m, cols, _I32_MAX),
                       axis=1, keepdims=True)
        ids.append(pick)
        bm = jnp.where(cols == pick, _NEG_INF, bm)
    bid_ref[...] = jnp.concatenate(ids, axis=1)


_SC_NC = 2
_SC_NS = 16
_SC_NW = _SC_NC * _SC_NS
_GATHER_N = Q * K
_G_PER_W = _GATHER_N // _SC_NW


def _sc_gather_body(table_hbm, idx_hbm, out_hbm, idx_v, rows_v, sem):
    wid = lax.axis_index("s") * _SC_NC + lax.axis_index("c")
    base = wid * _G_PER_W
    pltpu.sync_copy(idx_hbm.at[pl.ds(base, _G_PER_W)], idx_v)
    pltpu.async_copy(table_hbm.at[idx_v], rows_v, sem).wait()
    pltpu.sync_copy(rows_v, out_hbm.at[pl.ds(base, _G_PER_W)])


def _stage_d_body(cand_ref, bid_ref, idx_ref):
    v = cand_ref[...]
    bid = bid_ref[...]
    lane = jax.lax.broadcasted_iota(jnp.int32, (Q, K, SEG), 2)
    gcol = bid[:, :, None] * SEG + lane
    picks = []
    for _ in range(K):
        m = jnp.max(v, axis=(1, 2), keepdims=True)
        pick = jnp.min(jnp.where(v == m, gcol, _I32_MAX),
                       axis=(1, 2), keepdims=True)
        picks.append(pick.reshape(Q, 1))
        v = jnp.where(gcol == pick, _NEG_INF, v)
    idx_ref[...] = jnp.concatenate(picks, axis=1)


def _sc_gather_wide_body(table_hbm, idx_hbm, out_hbm, idx_v, rows_v, sem):
    wid = lax.axis_index("s") * _SC_NC + lax.axis_index("c")
    base = wid * _G_PER_W
    pltpu.sync_copy(idx_hbm.at[pl.ds(base, _G_PER_W)], idx_v)
    pltpu.async_copy(table_hbm.at[idx_v], rows_v, sem).wait()
    pltpu.sync_copy(rows_v, out_hbm.at[pl.ds(base, _G_PER_W)])


def _stage_e2_body(wide_ref, sub_ref, o_ref):
    g3 = wide_ref[...].reshape(Q, K, 4 * D)
    sub3 = sub_ref[...][:, :, None]
    out = jnp.zeros((Q, K, D), jnp.float32)
    for i in range(4):
        out = out + jnp.where(sub3 == i, g3[:, :, i * D:(i + 1) * D], 0.0)
    o_ref[...] = out


def kernel(query, keys, values, n_per_key):
    m_total = keys.shape[0]
    nchunks = -(-m_total // CHUNK)
    nb = nchunks * SEGS

    sims2, bmax3 = pl.pallas_call(
        functools.partial(_stage_a_body, m_total=m_total),
        grid=(nchunks,),
        in_specs=[
            pl.BlockSpec((Q, D), lambda i: (0, 0)),
            pl.BlockSpec((CHUNK, D), lambda i: (i, 0)),
        ],
        out_specs=[
            pl.BlockSpec((SEGS * Q, SEG), lambda i: (i, 0)),
            pl.BlockSpec((1, Q, SEGS), lambda i: (i, 0, 0)),
        ],
        out_shape=[
            jax.ShapeDtypeStruct((nb * Q, SEG), jnp.float32),
            jax.ShapeDtypeStruct((nchunks, Q, SEGS), jnp.float32),
        ],
        compiler_params=pltpu.CompilerParams(
            dimension_semantics=("parallel",)),
        interpret=_INTERPRET,
    )(query, keys)

    bmax = jnp.transpose(bmax3, (1, 0, 2)).reshape(Q, nb)
    bid = pl.pallas_call(
        functools.partial(_stage_b_body, nb=nb),
        out_shape=jax.ShapeDtypeStruct((Q, K), jnp.int32),
        interpret=_INTERPRET,
    )(bmax)

    row_idx = (bid * Q + jnp.arange(Q, dtype=jnp.int32)[:, None]
               ).reshape(Q * K)
    sc_gather = functools.partial(
        pl.kernel,
        mesh=plsc.VectorSubcoreMesh(core_axis_name="c", subcore_axis_name="s"),
        out_type=jax.ShapeDtypeStruct((_GATHER_N, SEG), jnp.float32),
        scratch_types=[
            pltpu.VMEM((_G_PER_W,), jnp.int32),
            pltpu.VMEM((_G_PER_W, SEG), jnp.float32),
            pltpu.SemaphoreType.DMA,
        ],
    )(_sc_gather_body)
    cand = sc_gather(sims2, row_idx)

    idx2 = pl.pallas_call(
        _stage_d_body,
        out_shape=jax.ShapeDtypeStruct((Q, K), jnp.int32),
        interpret=_INTERPRET,
    )(cand.reshape(Q, K, SEG), bid)

    values_wide = values.reshape(m_total // 4, 4 * D)
    idx_flat = jnp.clip(idx2.reshape(Q * K) + (n_per_key - K),
                        0, m_total - 1).astype(jnp.int32)
    sc_gather_wide = functools.partial(
        pl.kernel,
        mesh=plsc.VectorSubcoreMesh(core_axis_name="c", subcore_axis_name="s"),
        out_type=jax.ShapeDtypeStruct((_GATHER_N, 4 * D), jnp.float32),
        scratch_types=[
            pltpu.VMEM((_G_PER_W,), jnp.int32),
            pltpu.VMEM((_G_PER_W, 4 * D), jnp.float32),
            pltpu.SemaphoreType.DMA,
        ],
    )(_sc_gather_wide_body)
    wide = sc_gather_wide(values_wide, idx_flat // 4)

    recalled = pl.pallas_call(
        _stage_e2_body,
        out_shape=jax.ShapeDtypeStruct((Q, K, D), jnp.float32),
        interpret=_INTERPRET,
    )(wide, (idx_flat % 4).reshape(Q, K))

    return recalled


# speedup vs baseline: 1.1356x; 1.1356x over previous
import functools

import jax
import jax.numpy as jnp
from jax import lax
from jax.experimental import pallas as pl
from jax.experimental.pallas import tpu as pltpu
from jax.experimental.pallas import tpu_sc as plsc

_INTERPRET = False

Q = 64
D = 32
K = 8
CHUNK = 16384
SEG = 512
SEGS = CHUNK // SEG

_NEG_INF = float("-inf")
_I32_MAX = 2**31 - 1


def _stage_a_body(q_ref, k_ref, s_ref, b_ref, *, m_total):
    i = pl.program_id(0)
    q = q_ref[...]
    k = k_ref[...]
    s = jax.lax.dot_general(
        q, k, dimension_numbers=(((1,), (1,)), ((), ())),
        preferred_element_type=jnp.float32)
    col = i * CHUNK + jax.lax.broadcasted_iota(jnp.int32, (Q, CHUNK), 1)
    s = jnp.where(col < m_total, s, _NEG_INF)
    s3 = s.reshape(Q, SEGS, SEG)
    s_ref[...] = jnp.transpose(s3, (1, 0, 2)).reshape(SEGS * Q, SEG)
    b_ref[0] = jnp.max(s3, axis=2)


def _stage_b_body(bm_ref, bid_ref, *, nb):
    bm = bm_ref[...]
    cols = jax.lax.broadcasted_iota(jnp.int32, (Q, nb), 1)
    ids = []
    for _ in range(K):
        m = jnp.max(bm, axis=1, keepdims=True)
        pick = jnp.min(jnp.where(bm == m, cols, _I32_MAX),
                       axis=1, keepdims=True)
        ids.append(pick)
        bm = jnp.where(cols == pick, _NEG_INF, bm)
    bid_ref[...] = jnp.concatenate(ids, axis=1)


_SC_NC = 2
_SC_NS = 16
_SC_NW = _SC_NC * _SC_NS
_GATHER_N = Q * K
_G_PER_W = _GATHER_N // _SC_NW


def _sc_gather_body(table_hbm, idx_hbm, out_hbm, idx_v, rows_v, sem):
    wid = lax.axis_index("s") * _SC_NC + lax.axis_index("c")
    base = wid * _G_PER_W
    pltpu.sync_copy(idx_hbm.at[pl.ds(base, _G_PER_W)], idx_v)
    pltpu.async_copy(table_hbm.at[idx_v], rows_v, sem).wait()
    pltpu.sync_copy(rows_v, out_hbm.at[pl.ds(base, _G_PER_W)])


def _stage_d_body(cand_ref, bid_ref, idx_ref):
    v = cand_ref[...]
    bid = bid_ref[...]
    lane = jax.lax.broadcasted_iota(jnp.int32, (Q, K, SEG), 2)
    gcol = bid[:, :, None] * SEG + lane
    picks = []
    for _ in range(K):
        m = jnp.max(v, axis=(1, 2), keepdims=True)
        pick = jnp.min(jnp.where(v == m, gcol, _I32_MAX),
                       axis=(1, 2), keepdims=True)
        picks.append(pick.reshape(Q, 1))
        v = jnp.where(gcol == pick, _NEG_INF, v)
    idx_ref[...] = jnp.concatenate(picks, axis=1)


_E_SEMS = 8


def _stage_e_body(idx_ref, v_ref, o_ref, rows_v, sems):
    copies = []
    for i in range(_GATHER_N):
        cp = pltpu.make_async_copy(
            v_ref.at[pl.ds(idx_ref[i], 1), :],
            rows_v.at[pl.ds(i, 1), :],
            sems.at[i % _E_SEMS])
        cp.start()
        copies.append(cp)
    for cp in copies:
        cp.wait()
    o_ref[...] = rows_v[...].reshape(Q, K, D)


def kernel(query, keys, values, n_per_key):
    m_total = keys.shape[0]
    nchunks = -(-m_total // CHUNK)
    nb = nchunks * SEGS

    sims2, bmax3 = pl.pallas_call(
        functools.partial(_stage_a_body, m_total=m_total),
        grid=(nchunks,),
        in_specs=[
            pl.BlockSpec((Q, D), lambda i: (0, 0)),
            pl.BlockSpec((CHUNK, D), lambda i: (i, 0)),
        ],
        out_specs=[
            pl.BlockSpec((SEGS * Q, SEG), lambda i: (i, 0)),
            pl.BlockSpec((1, Q, SEGS), lambda i: (i, 0, 0)),
        ],
        out_shape=[
            jax.ShapeDtypeStruct((nb * Q, SEG), jnp.float32),
            jax.ShapeDtypeStruct((nchunks, Q, SEGS), jnp.float32),
        ],
        compiler_params=pltpu.CompilerParams(
            dimension_semantics=("parallel",)),
        interpret=_INTERPRET,
    )(query, keys)

    bmax = jnp.transpose(bmax3, (1, 0, 2)).reshape(Q, nb)
    bid = pl.pallas_call(
        functools.partial(_stage_b_body, nb=nb),
        out_shape=jax.ShapeDtypeStruct((Q, K), jnp.int32),
        interpret=_INTERPRET,
    )(bmax)

    row_idx = (bid * Q + jnp.arange(Q, dtype=jnp.int32)[:, None]
               ).reshape(Q * K)
    sc_gather = functools.partial(
        pl.kernel,
        mesh=plsc.VectorSubcoreMesh(core_axis_name="c", subcore_axis_name="s"),
        out_type=jax.ShapeDtypeStruct((_GATHER_N, SEG), jnp.float32),
        scratch_types=[
            pltpu.VMEM((_G_PER_W,), jnp.int32),
            pltpu.VMEM((_G_PER_W, SEG), jnp.float32),
            pltpu.SemaphoreType.DMA,
        ],
    )(_sc_gather_body)
    cand = sc_gather(sims2, row_idx)

    idx2 = pl.pallas_call(
        _stage_d_body,
        out_shape=jax.ShapeDtypeStruct((Q, K), jnp.int32),
        interpret=_INTERPRET,
    )(cand.reshape(Q, K, SEG), bid)

    idx_flat = jnp.clip(idx2.reshape(Q * K) + (n_per_key - K),
                        0, m_total - 1).astype(jnp.int32)
    recalled = pl.pallas_call(
        _stage_e_body,
        in_specs=[
            pl.BlockSpec(memory_space=pltpu.SMEM),
            pl.BlockSpec(memory_space=pl.ANY),
        ],
        out_specs=pl.BlockSpec(memory_space=pltpu.VMEM),
        out_shape=jax.ShapeDtypeStruct((Q, K, D), jnp.float32),
        scratch_shapes=[
            pltpu.VMEM((_GATHER_N, D), jnp.float32),
            pltpu.SemaphoreType.DMA((_E_SEMS,)),
        ],
        interpret=_INTERPRET,
    )(idx_flat, values)

    return recalled
